# baseline (speedup 1.0000x reference)
import functools

import jax
import jax.numpy as jnp
from jax import lax
from jax.experimental import pallas as pl
from jax.experimental.pallas import tpu as pltpu
from jax.experimental.pallas import tpu_sc as plsc

N = 10000
E = 320000
D = 128
D_EDGE = 16

NPAD = 10240
ROWS_PER_TILE = NPAD // 16
NW = 32
E_PER_W = E // NW
EC = 40
CHUNKS = E_PER_W // EC


_MESH = dict(core_axis_name="c", subcore_axis_name="s")


def _zero_rows(buf, nrows):
    def body(i, _):
        for j in range(D // 16):
            buf[i, pl.ds(j * 16, 16)] = jnp.zeros((16,), jnp.float32)
        return 0
    lax.fori_loop(0, nrows, body, 0)


def _sc_aggregate(P, Q, src, dst):
    NPAIR = CHUNKS // 2
    LOOP_PAIRS = (NPAIR - 1) // 2 * 2

    @functools.partial(
        pl.kernel,
        out_type=jax.ShapeDtypeStruct((2, NPAD, D), jnp.float32),
        mesh=plsc.VectorSubcoreMesh(**_MESH),
        scratch_types=[
            pltpu.VMEM((2, EC), jnp.int32),
            pltpu.VMEM((2, EC), jnp.int32),
            pltpu.VMEM((2, EC), jnp.int32),
            pltpu.VMEM((2, EC), jnp.int32),
            pltpu.VMEM((EC, D), jnp.float32),
            pltpu.VMEM((EC, D), jnp.float32),
            pltpu.VMEM((EC // 8, 8, D // 2), jnp.int32),
            pltpu.VMEM((EC // 8, 8, D // 2), jnp.int32),
            pltpu.VMEM((EC, D), jnp.float32),
            pltpu.VMEM((EC, D), jnp.float32),
            pltpu.VMEM_SHARED((NPAD, D), jnp.float32),
            pltpu.SemaphoreType.DMA,
            pltpu.SemaphoreType.DMA,
            pltpu.SemaphoreType.DMA,
            pltpu.SemaphoreType.DMA,
            pltpu.SemaphoreType.DMA,
            pltpu.SemaphoreType.DMA,
            pltpu.SemaphoreType.DMA,
            pltpu.SemaphoreType.DMA,
        ],
    )
    def k(p_hbm, q_hbm, src_hbm, dst_hbm, agg_out,
          si0, si1, di0, di1, prow0, prow1, qrow0, qrow1, fo0, fo1, agg_sh,
          is0, is1, gs0, gs1, qs0, qs1, ss0, ss1):
        c = lax.axis_index("c")
        s = lax.axis_index("s")
        wid = c * 16 + s
        sidx = (si0, si1)
        didx = (di0, di1)
        isem = (is0, is1)
        prow = (prow0, prow1)
        qrow = (qrow0, qrow1)
        gsem = (gs0, gs1)
        qsem = (qs0, qs1)
        ssem = (ss0, ss1)
        fout = (fo0, fo1)

        _zero_rows(fo0, EC)
        for kk in range(ROWS_PER_TILE // EC):
            pltpu.async_copy(fo0, agg_sh.at[pl.ds(s * ROWS_PER_TILE + kk * EC, EC)],
                             ss0)
        for kk in range(ROWS_PER_TILE // EC):
            pltpu.make_async_copy(fo0, agg_sh.at[pl.ds(0, EC)], ss0).wait()
        plsc.subcore_barrier()

        qbase = wid * E_PER_W

        def issue_idx(p, slot):
            pltpu.async_copy(src_hbm.at[wid, p], sidx[slot], isem[slot])
            pltpu.async_copy(dst_hbm.at[wid, p], didx[slot], isem[slot])

        def wait_idx(p, slot):
            pltpu.make_async_copy(src_hbm.at[wid, p], sidx[slot],
                                  isem[slot]).wait()
            pltpu.make_async_copy(dst_hbm.at[wid, p], didx[slot],
                                  isem[slot]).wait()

        def issue_in(g, slot, b):
            pltpu.async_copy(p_hbm.at[sidx[slot].at[g % 2]], prow[b], gsem[b])
            pltpu.async_copy(q_hbm.at[wid, g], qrow[b], qsem[b])

        def wait_in(g, b):
            pltpu.make_async_copy(q_hbm.at[wid, g], qrow[b], qsem[b]).wait()
            pltpu.make_async_copy(p_hbm.at[sidx[0].at[0]], prow[b],
                                  gsem[b]).wait()

        def issue_scatter(g, slot, sb):
            pltpu.async_copy(fout[sb], agg_sh.at[didx[slot].at[g % 2]],
                             ssem[sb], add=True)

        def drain_scatter(sb):
            pltpu.make_async_copy(fout[sb], agg_sh.at[didx[0].at[0]],
                                  ssem[sb]).wait()

        def compute(b):
            def row_body(i3, _):
                for r in range(8):
                    row = i3 * 8 + r
                    for j in range(D // 32):
                        qw = qrow[b][i3, r, pl.ds(j * 16, 16)]
                        qlo = lax.bitcast_convert_type(qw << 16, jnp.float32)
                        qhi = lax.bitcast_convert_type(
                            qw & jnp.int32(-65536), jnp.float32)
                        lo_sl = pl.ds(j * 32, 16)
                        hi_sl = pl.ds(j * 32 + 16, 16)
                        fout[b][row, lo_sl] = jnp.maximum(
                            prow[b][row, lo_sl] + qlo, 0.0)
                        fout[b][row, hi_sl] = jnp.maximum(
                            prow[b][row, hi_sl] + qhi, 0.0)
                return 0
            lax.fori_loop(0, EC // 8, row_body, 0)

        def chunk_step(g, slot, b, first):
            wait_in(g, b)
            if not first:
                drain_scatter(b)
            compute(b)
            issue_scatter(g, slot, b)
            issue_in(g + 2, slot ^ 1, b)

        issue_idx(0, 0)
        issue_idx(1, 1)
        wait_idx(0, 0)
        issue_in(0, 0, 0)
        issue_in(1, 0, 1)

        wait_idx(1, 1)
        chunk_step(0, 0, 0, first=True)
        chunk_step(1, 0, 1, first=True)
        issue_idx(2, 0)

        def duo_body(i, _):
            for pp in range(2):
                p = 2 * i + 1 + pp
                slot = pp ^ 1
                wait_idx(p + 1, slot ^ 1)
                for b in range(2):
                    g = 2 * p + b
                    chunk_step(g, slot, b, first=False)

                @pl.when(p + 2 < NPAIR)
                def _():
                    issue_idx(p + 2, slot)
            return 0
        lax.fori_loop(0, (NPAIR - 3) // 2, duo_body, 0)

        wait_idx(NPAIR - 1, 0)
        for b in range(2):
            chunk_step(2 * (NPAIR - 2) + b, 1, b, first=False)

        tp = NPAIR - 1
        for b in range(2):
            g = 2 * tp + b
            wait_in(g, b)
            drain_scatter(b)
            compute(b)
            issue_scatter(g, 0, b)
        drain_scatter(0)
        drain_scatter(1)

        plsc.subcore_barrier()
        stripe = pl.ds(s * ROWS_PER_TILE, ROWS_PER_TILE)
        pltpu.sync_copy(agg_sh.at[stripe], agg_out.at[c, stripe])

    return k(P, Q.reshape(NW, CHUNKS, EC // 8, 8, D // 2),
             src.reshape(NW, NPAIR, 2, EC),
             dst.reshape(NW, NPAIR, 2, EC))


def _sc_degree(dst):
    @functools.partial(
        pl.kernel,
        out_type=jax.ShapeDtypeStruct((2, NPAD, D), jnp.float32),
        mesh=plsc.VectorSubcoreMesh(**_MESH),
        scratch_types=[
            pltpu.VMEM((CHUNKS, EC), jnp.int32),
            pltpu.VMEM((EC, D), jnp.float32),
            pltpu.VMEM_SHARED((NPAD, D), jnp.float32),
            pltpu.SemaphoreType.DMA,
            pltpu.SemaphoreType.DMA,
        ],
    )
    def k(dst_hbm, deg_out, didx, rows, deg_sh, isem, ssem):
        c = lax.axis_index("c")
        s = lax.axis_index("s")
        wid = c * 16 + s

        pltpu.async_copy(dst_hbm.at[wid], didx, isem)

        _zero_rows(rows, EC)
        for kk in range(ROWS_PER_TILE // EC):
            pltpu.async_copy(rows, deg_sh.at[pl.ds(s * ROWS_PER_TILE + kk * EC, EC)],
                             ssem)
        for kk in range(ROWS_PER_TILE // EC):
            pltpu.make_async_copy(rows, deg_sh.at[pl.ds(0, EC)], ssem).wait()

        def ofill(i, _):
            for j in range(D // 16):
                rows[i, pl.ds(j * 16, 16)] = jnp.ones((16,), jnp.float32)
            return 0
        lax.fori_loop(0, EC, ofill, 0)
        pltpu.make_async_copy(dst_hbm.at[wid], didx, isem).wait()
        plsc.subcore_barrier()

        def fire(t, _):
            pltpu.async_copy(rows, deg_sh.at[didx.at[t]], ssem, add=True)
            return 0
        lax.fori_loop(0, CHUNKS, fire, 0)

        def drain(t, _):
            pltpu.make_async_copy(rows, deg_sh.at[didx.at[0]], ssem).wait()
            return 0
        lax.fori_loop(0, CHUNKS, drain, 0)

        plsc.subcore_barrier()
        stripe = pl.ds(s * ROWS_PER_TILE, ROWS_PER_TILE)
        pltpu.sync_copy(deg_sh.at[stripe], deg_out.at[c, stripe])

    return k(dst.reshape(NW, CHUNKS, EC))


_BN = 1000
_BE = 4000


def _p_body(x_ref, w_ref, o_ref):
    o_ref[...] = jnp.dot(x_ref[...], w_ref[...],
                         preferred_element_type=jnp.float32)


def _p_matmul(x, w):
    return pl.pallas_call(
        _p_body,
        grid=(N // _BN,),
        in_specs=[pl.BlockSpec((_BN, D), lambda i: (i, 0)),
                  pl.BlockSpec((D, D), lambda i: (0, 0))],
        out_specs=pl.BlockSpec((_BN, D), lambda i: (i, 0)),
        out_shape=jax.ShapeDtypeStruct((N, D), jnp.float32),
    )(x, w)


def _pack_bf16_pairs(even_f32, odd_f32):
    be = lax.bitcast_convert_type(
        even_f32.astype(jnp.bfloat16).astype(jnp.float32), jnp.uint32)
    bo = lax.bitcast_convert_type(
        odd_f32.astype(jnp.bfloat16).astype(jnp.float32), jnp.uint32)
    packed = jnp.bitwise_or(be >> 16,
                            jnp.bitwise_and(bo, jnp.uint32(0xFFFF0000)))
    return lax.bitcast_convert_type(packed, jnp.int32)


def _q_body(ea_ref, w1e_ref, w1o_ref, b1e_ref, b1o_ref,
            w2e_ref, w2o_ref, b2e_ref, b2o_ref, q1_ref, q2_ref):
    ea = ea_ref[...]
    q1e = jnp.dot(ea, w1e_ref[...], preferred_element_type=jnp.float32) + b1e_ref[...]
    q1o = jnp.dot(ea, w1o_ref[...], preferred_element_type=jnp.float32) + b1o_ref[...]
    q1_ref[...] = _pack_bf16_pairs(q1e, q1o)
    q2e = jnp.dot(ea, w2e_ref[...], preferred_element_type=jnp.float32) + b2e_ref[...]
    q2o = jnp.dot(ea, w2o_ref[...], preferred_element_type=jnp.float32) + b2o_ref[...]
    q2_ref[...] = _pack_bf16_pairs(q2e, q2o)


def _q_matmul(edge_attr, w1e, w1o, b1e, b1o, w2e, w2o, b2e, b2o):
    wspec = pl.BlockSpec((D_EDGE, D // 2), lambda i: (0, 0))
    bspec = pl.BlockSpec((1, D // 2), lambda i: (0, 0))
    return pl.pallas_call(
        _q_body,
        grid=(E // _BE,),
        in_specs=[pl.BlockSpec((_BE, D_EDGE), lambda i: (i, 0)),
                  wspec, wspec, bspec, bspec, wspec, wspec, bspec, bspec],
        out_specs=[pl.BlockSpec((_BE, D // 2), lambda i: (i, 0)),
                   pl.BlockSpec((_BE, D // 2), lambda i: (i, 0))],
        out_shape=[jax.ShapeDtypeStruct((E, D // 2), jnp.int32),
                   jax.ShapeDtypeStruct((E, D // 2), jnp.int32)],
    )(edge_attr, w1e, w1o, b1e.reshape(1, D // 2), b1o.reshape(1, D // 2),
      w2e, w2o, b2e.reshape(1, D // 2), b2o.reshape(1, D // 2))


def _update_common(x_ref, a0_ref, a1_ref, d0_ref, d1_ref,
                   wux_ref, wua_ref, bu_ref, bias_ref):
    deg = d0_ref[...][:, :1] + d1_ref[...][:, :1]
    agg = (a0_ref[...] + a1_ref[...]) / jnp.maximum(deg, 1.0)
    out = (jnp.dot(x_ref[...], wux_ref[...], preferred_element_type=jnp.float32)
           + jnp.dot(agg, wua_ref[...], preferred_element_type=jnp.float32)
           + bu_ref[...])
    nrm = jnp.sqrt(jnp.sum(out * out, axis=-1, keepdims=True))
    return out / jnp.maximum(nrm, 1e-12) + bias_ref[...]


def _upd1_body(x_ref, a0_ref, a1_ref, d0_ref, d1_ref,
               wux_ref, wua_ref, bu_ref, bias_ref, wnext_ref,
               h_ref, p2_ref):
    out = _update_common(x_ref, a0_ref, a1_ref, d0_ref, d1_ref,
                         wux_ref, wua_ref, bu_ref, bias_ref)
    h = jnp.maximum(out, 0.0)
    h_ref[...] = h
    p2_ref[...] = jnp.dot(h, wnext_ref[...], preferred_element_type=jnp.float32)


def _upd2_body(x_ref, a0_ref, a1_ref, d0_ref, d1_ref,
               wux_ref, wua_ref, bu_ref, bias_ref, o_ref):
    o_ref[...] = _update_common(x_ref, a0_ref, a1_ref, d0_ref, d1_ref,
                                wux_ref, wua_ref, bu_ref, bias_ref)


def _upd_specs():
    return [pl.BlockSpec((_BN, D), lambda i: (i, 0)),
            pl.BlockSpec((_BN, D), lambda i: (i, 0)),
            pl.BlockSpec((_BN, D), lambda i: (i, 0)),
            pl.BlockSpec((_BN, 8), lambda i: (i, 0)),
            pl.BlockSpec((_BN, 8), lambda i: (i, 0)),
            pl.BlockSpec((D, D), lambda i: (0, 0)),
            pl.BlockSpec((D, D), lambda i: (0, 0)),
            pl.BlockSpec((1, D), lambda i: (0, 0)),
            pl.BlockSpec((1, D), lambda i: (0, 0))]


def _update1(x, aggp, d0, d1, wu, bu, bias, wnext):
    return pl.pallas_call(
        _upd1_body,
        grid=(N // _BN,),
        in_specs=_upd_specs() + [pl.BlockSpec((D, D), lambda i: (0, 0))],
        out_specs=[pl.BlockSpec((_BN, D), lambda i: (i, 0)),
                   pl.BlockSpec((_BN, D), lambda i: (i, 0))],
        out_shape=[jax.ShapeDtypeStruct((N, D), jnp.float32),
                   jax.ShapeDtypeStruct((N, D), jnp.float32)],
    )(x, aggp[0, :N], aggp[1, :N], d0, d1,
      wu[:D], wu[D:], bu.reshape(1, D), bias.reshape(1, D), wnext)


def _update2(x, aggp, d0, d1, wu, bu, bias):
    return pl.pallas_call(
        _upd2_body,
        grid=(N // _BN,),
        in_specs=_upd_specs(),
        out_specs=pl.BlockSpec((_BN, D), lambda i: (i, 0)),
        out_shape=jax.ShapeDtypeStruct((N, D), jnp.float32),
    )(x, aggp[0, :N], aggp[1, :N], d0, d1,
      wu[:D], wu[D:], bu.reshape(1, D), bias.reshape(1, D))


def kernel(x, edge_index, edge_attr,
           W1m, b1m, W1u, b1u, bias1, W2m, b2m, W2u, b2u, bias2):
    src = edge_index[0]
    dst = edge_index[1]

    t = jnp.arange(D // 2)
    EV = (t // 16) * 32 + t % 16
    OD = EV + 16

    P1 = _p_matmul(x, W1m[:D])
    Q1, Q2 = _q_matmul(edge_attr, W1m[D:][:, EV], W1m[D:][:, OD],
                       b1m[EV], b1m[OD],
                       W2m[D:][:, EV], W2m[D:][:, OD], b2m[EV], b2m[OD])

    degp = _sc_degree(dst)
    d0 = degp[0, :N, :8]
    d1 = degp[1, :N, :8]

    aggp1 = _sc_aggregate(P1, Q1, src, dst)
    h, P2 = _update1(x, aggp1, d0, d1, W1u, b1u, bias1, W2m[:D])

    aggp2 = _sc_aggregate(P2, Q2, src, dst)
    out = _update2(h, aggp2, d0, d1, W2u, b2u, bias2)
    return out


# speedup vs baseline: 1.0173x; 1.0173x over previous
import functools

import jax
import jax.numpy as jnp
from jax import lax
from jax.experimental import pallas as pl
from jax.experimental.pallas import tpu as pltpu
from jax.experimental.pallas import tpu_sc as plsc

N = 10000
E = 320000
D = 128
D_EDGE = 16

NPAD = 10240
ROWS_PER_TILE = NPAD // 16
NW = 32
E_PER_W = E // NW
EC = 40
CHUNKS = E_PER_W // EC


_MESH = dict(core_axis_name="c", subcore_axis_name="s")


def _zero_rows(buf, nrows):
    def body(i, _):
        for j in range(D // 16):
            buf[i, pl.ds(j * 16, 16)] = jnp.zeros((16,), jnp.float32)
        return 0
    lax.fori_loop(0, nrows, body, 0)


def _sc_aggregate(P, Q, src, dst):
    NPAIR = CHUNKS // 2
    LOOP_PAIRS = (NPAIR - 1) // 2 * 2

    @functools.partial(
        pl.kernel,
        out_type=jax.ShapeDtypeStruct((2, NPAD, D), jnp.float32),
        mesh=plsc.VectorSubcoreMesh(**_MESH),
        scratch_types=[
            pltpu.VMEM((2, EC), jnp.int32),
            pltpu.VMEM((2, EC), jnp.int32),
            pltpu.VMEM((2, EC), jnp.int32),
            pltpu.VMEM((2, EC), jnp.int32),
            pltpu.VMEM((EC, D), jnp.float32),
            pltpu.VMEM((EC, D), jnp.float32),
            pltpu.VMEM((EC, D), jnp.float32),
            pltpu.VMEM((EC, D), jnp.float32),
            pltpu.VMEM((EC, D), jnp.float32),
            pltpu.VMEM((EC, D), jnp.float32),
            pltpu.VMEM_SHARED((NPAD, D), jnp.float32),
            pltpu.SemaphoreType.DMA,
            pltpu.SemaphoreType.DMA,
            pltpu.SemaphoreType.DMA,
            pltpu.SemaphoreType.DMA,
            pltpu.SemaphoreType.DMA,
            pltpu.SemaphoreType.DMA,
            pltpu.SemaphoreType.DMA,
            pltpu.SemaphoreType.DMA,
        ],
    )
    def k(p_hbm, q_hbm, src_hbm, dst_hbm, agg_out,
          si0, si1, di0, di1, prow0, prow1, qrow0, qrow1, fo0, fo1, agg_sh,
          is0, is1, gs0, gs1, qs0, qs1, ss0, ss1):
        c = lax.axis_index("c")
        s = lax.axis_index("s")
        wid = c * 16 + s
        sidx = (si0, si1)
        didx = (di0, di1)
        isem = (is0, is1)
        prow = (prow0, prow1)
        qrow = (qrow0, qrow1)
        gsem = (gs0, gs1)
        qsem = (qs0, qs1)
        ssem = (ss0, ss1)
        fout = (fo0, fo1)

        _zero_rows(fo0, EC)
        for kk in range(ROWS_PER_TILE // EC):
            pltpu.async_copy(fo0, agg_sh.at[pl.ds(s * ROWS_PER_TILE + kk * EC, EC)],
                             ss0)
        for kk in range(ROWS_PER_TILE // EC):
            pltpu.make_async_copy(fo0, agg_sh.at[pl.ds(0, EC)], ss0).wait()
        plsc.subcore_barrier()

        qbase = wid * E_PER_W

        def issue_idx(p, slot):
            pltpu.async_copy(src_hbm.at[wid, p], sidx[slot], isem[slot])
            pltpu.async_copy(dst_hbm.at[wid, p], didx[slot], isem[slot])

        def wait_idx(p, slot):
            pltpu.make_async_copy(src_hbm.at[wid, p], sidx[slot],
                                  isem[slot]).wait()
            pltpu.make_async_copy(dst_hbm.at[wid, p], didx[slot],
                                  isem[slot]).wait()

        def issue_in(g, slot, b):
            pltpu.async_copy(p_hbm.at[sidx[slot].at[g % 2]], prow[b], gsem[b])
            pltpu.async_copy(q_hbm.at[pl.ds(qbase + g * EC, EC)],
                             qrow[b], qsem[b])

        def wait_in(g, b):
            pltpu.make_async_copy(q_hbm.at[pl.ds(qbase + g * EC, EC)],
                                  qrow[b], qsem[b]).wait()
            pltpu.make_async_copy(p_hbm.at[sidx[0].at[0]], prow[b],
                                  gsem[b]).wait()

        def issue_scatter(g, slot, sb):
            pltpu.async_copy(fout[sb], agg_sh.at[didx[slot].at[g % 2]],
                             ssem[sb], add=True)

        def drain_scatter(sb):
            pltpu.make_async_copy(fout[sb], agg_sh.at[didx[0].at[0]],
                                  ssem[sb]).wait()

        def compute(b):
            def row_body(i, _):
                for j in range(D // 16):
                    sl = pl.ds(j * 16, 16)
                    fout[b][i, sl] = jnp.maximum(
                        prow[b][i, sl] + qrow[b][i, sl], 0.0)
                return 0
            lax.fori_loop(0, EC, row_body, 0)

        def chunk_step(g, slot, b, first):
            wait_in(g, b)
            if not first:
                drain_scatter(b)
            compute(b)
            issue_scatter(g, slot, b)
            issue_in(g + 2, slot ^ 1, b)

        issue_idx(0, 0)
        issue_idx(1, 1)
        wait_idx(0, 0)
        issue_in(0, 0, 0)
        issue_in(1, 0, 1)

        wait_idx(1, 1)
        chunk_step(0, 0, 0, first=True)
        chunk_step(1, 0, 1, first=True)
        issue_idx(2, 0)

        def duo_body(i, _):
            for pp in range(2):
                p = 2 * i + 1 + pp
                slot = pp ^ 1
                wait_idx(p + 1, slot ^ 1)
                for b in range(2):
                    g = 2 * p + b
                    chunk_step(g, slot, b, first=False)

                @pl.when(p + 2 < NPAIR)
                def _():
                    issue_idx(p + 2, slot)
            return 0
        lax.fori_loop(0, (NPAIR - 3) // 2, duo_body, 0)

        wait_idx(NPAIR - 1, 0)
        for b in range(2):
            chunk_step(2 * (NPAIR - 2) + b, 1, b, first=False)

        tp = NPAIR - 1
        for b in range(2):
            g = 2 * tp + b
            wait_in(g, b)
            drain_scatter(b)
            compute(b)
            issue_scatter(g, 0, b)
        drain_scatter(0)
        drain_scatter(1)

        plsc.subcore_barrier()
        stripe = pl.ds(s * ROWS_PER_TILE, ROWS_PER_TILE)
        pltpu.sync_copy(agg_sh.at[stripe], agg_out.at[c, stripe])

    return k(P, Q, src.reshape(NW, NPAIR, 2, EC),
             dst.reshape(NW, NPAIR, 2, EC))


def _sc_degree(dst):
    @functools.partial(
        pl.kernel,
        out_type=jax.ShapeDtypeStruct((2, NPAD, D), jnp.float32),
        mesh=plsc.VectorSubcoreMesh(**_MESH),
        scratch_types=[
            pltpu.VMEM((CHUNKS, EC), jnp.int32),
            pltpu.VMEM((EC, D), jnp.float32),
            pltpu.VMEM_SHARED((NPAD, D), jnp.float32),
            pltpu.SemaphoreType.DMA,
            pltpu.SemaphoreType.DMA,
        ],
    )
    def k(dst_hbm, deg_out, didx, rows, deg_sh, isem, ssem):
        c = lax.axis_index("c")
        s = lax.axis_index("s")
        wid = c * 16 + s

        pltpu.async_copy(dst_hbm.at[wid], didx, isem)

        _zero_rows(rows, EC)
        for kk in range(ROWS_PER_TILE // EC):
            pltpu.async_copy(rows, deg_sh.at[pl.ds(s * ROWS_PER_TILE + kk * EC, EC)],
                             ssem)
        for kk in range(ROWS_PER_TILE // EC):
            pltpu.make_async_copy(rows, deg_sh.at[pl.ds(0, EC)], ssem).wait()

        def ofill(i, _):
            for j in range(D // 16):
                rows[i, pl.ds(j * 16, 16)] = jnp.ones((16,), jnp.float32)
            return 0
        lax.fori_loop(0, EC, ofill, 0)
        pltpu.make_async_copy(dst_hbm.at[wid], didx, isem).wait()
        plsc.subcore_barrier()

        def fire(t, _):
            pltpu.async_copy(rows, deg_sh.at[didx.at[t]], ssem, add=True)
            return 0
        lax.fori_loop(0, CHUNKS, fire, 0)

        def drain(t, _):
            pltpu.make_async_copy(rows, deg_sh.at[didx.at[0]], ssem).wait()
            return 0
        lax.fori_loop(0, CHUNKS, drain, 0)

        plsc.subcore_barrier()
        stripe = pl.ds(s * ROWS_PER_TILE, ROWS_PER_TILE)
        pltpu.sync_copy(deg_sh.at[stripe], deg_out.at[c, stripe])

    return k(dst.reshape(NW, CHUNKS, EC))


_BN = 1000
_BE = 4000


def _p_body(x_ref, w_ref, o_ref):
    o_ref[...] = jnp.dot(x_ref[...], w_ref[...],
                         preferred_element_type=jnp.float32)


def _p_matmul(x, w):
    return pl.pallas_call(
        _p_body,
        grid=(N // _BN,),
        in_specs=[pl.BlockSpec((_BN, D), lambda i: (i, 0)),
                  pl.BlockSpec((D, D), lambda i: (0, 0))],
        out_specs=pl.BlockSpec((_BN, D), lambda i: (i, 0)),
        out_shape=jax.ShapeDtypeStruct((N, D), jnp.float32),
    )(x, w)


def _q_body(ea_ref, w1_ref, b1_ref, w2_ref, b2_ref, q1_ref, q2_ref):
    ea = ea_ref[...]
    q1_ref[...] = jnp.dot(ea, w1_ref[...],
                          preferred_element_type=jnp.float32) + b1_ref[...]
    q2_ref[...] = jnp.dot(ea, w2_ref[...],
                          preferred_element_type=jnp.float32) + b2_ref[...]


def _q_matmul(edge_attr, w1, b1, w2, b2):
    return pl.pallas_call(
        _q_body,
        grid=(E // _BE,),
        in_specs=[pl.BlockSpec((_BE, D_EDGE), lambda i: (i, 0)),
                  pl.BlockSpec((D_EDGE, D), lambda i: (0, 0)),
                  pl.BlockSpec((1, D), lambda i: (0, 0)),
                  pl.BlockSpec((D_EDGE, D), lambda i: (0, 0)),
                  pl.BlockSpec((1, D), lambda i: (0, 0))],
        out_specs=[pl.BlockSpec((_BE, D), lambda i: (i, 0)),
                   pl.BlockSpec((_BE, D), lambda i: (i, 0))],
        out_shape=[jax.ShapeDtypeStruct((E, D), jnp.float32),
                   jax.ShapeDtypeStruct((E, D), jnp.float32)],
    )(edge_attr, w1, b1.reshape(1, D), w2, b2.reshape(1, D))


def _update_common(x_ref, a0_ref, a1_ref, d0_ref, d1_ref,
                   wux_ref, wua_ref, bu_ref, bias_ref):
    deg = d0_ref[...][:, :1] + d1_ref[...][:, :1]
    agg = (a0_ref[...] + a1_ref[...]) / jnp.maximum(deg, 1.0)
    out = (jnp.dot(x_ref[...], wux_ref[...], preferred_element_type=jnp.float32)
           + jnp.dot(agg, wua_ref[...], preferred_element_type=jnp.float32)
           + bu_ref[...])
    nrm = jnp.sqrt(jnp.sum(out * out, axis=-1, keepdims=True))
    return out / jnp.maximum(nrm, 1e-12) + bias_ref[...]


def _upd1_body(x_ref, a0_ref, a1_ref, d0_ref, d1_ref,
               wux_ref, wua_ref, bu_ref, bias_ref, wnext_ref,
               h_ref, p2_ref):
    out = _update_common(x_ref, a0_ref, a1_ref, d0_ref, d1_ref,
                         wux_ref, wua_ref, bu_ref, bias_ref)
    h = jnp.maximum(out, 0.0)
    h_ref[...] = h
    p2_ref[...] = jnp.dot(h, wnext_ref[...], preferred_element_type=jnp.float32)


def _upd2_body(x_ref, a0_ref, a1_ref, d0_ref, d1_ref,
               wux_ref, wua_ref, bu_ref, bias_ref, o_ref):
    o_ref[...] = _update_common(x_ref, a0_ref, a1_ref, d0_ref, d1_ref,
                                wux_ref, wua_ref, bu_ref, bias_ref)


def _upd_specs():
    return [pl.BlockSpec((_BN, D), lambda i: (i, 0)),
            pl.BlockSpec((_BN, D), lambda i: (i, 0)),
            pl.BlockSpec((_BN, D), lambda i: (i, 0)),
            pl.BlockSpec((_BN, 8), lambda i: (i, 0)),
            pl.BlockSpec((_BN, 8), lambda i: (i, 0)),
            pl.BlockSpec((D, D), lambda i: (0, 0)),
            pl.BlockSpec((D, D), lambda i: (0, 0)),
            pl.BlockSpec((1, D), lambda i: (0, 0)),
            pl.BlockSpec((1, D), lambda i: (0, 0))]


def _update1(x, aggp, d0, d1, wu, bu, bias, wnext):
    return pl.pallas_call(
        _upd1_body,
        grid=(N // _BN,),
        in_specs=_upd_specs() + [pl.BlockSpec((D, D), lambda i: (0, 0))],
        out_specs=[pl.BlockSpec((_BN, D), lambda i: (i, 0)),
                   pl.BlockSpec((_BN, D), lambda i: (i, 0))],
        out_shape=[jax.ShapeDtypeStruct((N, D), jnp.float32),
                   jax.ShapeDtypeStruct((N, D), jnp.float32)],
    )(x, aggp[0, :N], aggp[1, :N], d0, d1,
      wu[:D], wu[D:], bu.reshape(1, D), bias.reshape(1, D), wnext)


def _update2(x, aggp, d0, d1, wu, bu, bias):
    return pl.pallas_call(
        _upd2_body,
        grid=(N // _BN,),
        in_specs=_upd_specs(),
        out_specs=pl.BlockSpec((_BN, D), lambda i: (i, 0)),
        out_shape=jax.ShapeDtypeStruct((N, D), jnp.float32),
    )(x, aggp[0, :N], aggp[1, :N], d0, d1,
      wu[:D], wu[D:], bu.reshape(1, D), bias.reshape(1, D))


def kernel(x, edge_index, edge_attr,
           W1m, b1m, W1u, b1u, bias1, W2m, b2m, W2u, b2u, bias2):
    src = edge_index[0]
    dst = edge_index[1]

    P1 = _p_matmul(x, W1m[:D])
    Q1, Q2 = _q_matmul(edge_attr, W1m[D:], b1m, W2m[D:], b2m)

    degp = _sc_degree(dst)
    d0 = degp[0, :N, :8]
    d1 = degp[1, :N, :8]

    aggp1 = _sc_aggregate(P1, Q1, src, dst)
    h, P2 = _update1(x, aggp1, d0, d1, W1u, b1u, bias1, W2m[:D])

    aggp2 = _sc_aggregate(P2, Q2, src, dst)
    out = _update2(h, aggp2, d0, d1, W2u, b2u, bias2)
    return out
